# baseline (speedup 1.0000x reference)
import functools

import jax
import jax.numpy as jnp
from jax import lax
from jax.experimental import pallas as pl
from jax.experimental.pallas import tpu as pltpu
from jax.experimental.pallas import tpu_sc as plsc

LANES = 16
NC = 2
NS = 16
NW = NC * NS

D = 32
N_ROWS_PAD = 1000448
TBL_FLAT = N_ROWS_PAD * D

CW = 512
N_FULL = 1953
TAIL_I0 = N_FULL * CW
TAIL_N = 65

CHUNK = 1280
K = CHUNK // 128

_DNUMS = lax.GatherDimensionNumbers(
    offset_dims=(), collapsed_slice_dims=(0,), start_index_map=(0,)
)


def _format_body(wt_hbm, out_hbm, in0_v, in1_v, o0_v, o1_v,
                 sem_i0, sem_i1, sem_o0, sem_o1):
    wid = lax.axis_index("s") * NC + lax.axis_index("c")
    iota = lax.iota(jnp.int32, LANES)
    in_bufs = (in0_v, in1_v)
    out_bufs = (o0_v, o1_v)
    sems_i = (sem_i0, sem_i1)
    sems_o = (sem_o0, sem_o1)
    n_steps = 62

    def fire_in(c, b):
        i0 = pl.multiple_of(c * CW, 128)
        pltpu.async_copy(
            wt_hbm.at[:, pl.ds(i0, CW)], in_bufs[b].at[:, pl.ds(0, CW)],
            sems_i[b],
        )

    def drain_in(c, b):
        i0 = pl.multiple_of(c * CW, 128)
        pltpu.make_async_copy(
            wt_hbm.at[:, pl.ds(i0, CW)], in_bufs[b].at[:, pl.ds(0, CW)],
            sems_i[b],
        ).wait()

    def fire_out(c, b):
        r0 = pl.multiple_of(c * (CW * D), 8)
        pltpu.async_copy(
            out_bufs[b], out_hbm.at[pl.ds(r0, CW * D)], sems_o[b]
        )

    def drain_out(c, b):
        r0 = pl.multiple_of(c * (CW * D), 8)
        pltpu.make_async_copy(
            out_bufs[b], out_hbm.at[pl.ds(r0, CW * D)], sems_o[b]
        ).wait()

    rows_lo = iota
    rows_hi = iota + LANES

    def transpose_n(b, n_items):
        def body(i, carry):
            col = jnp.full((LANES,), 0, jnp.int32) + i
            lo = plsc.load_gather(in_bufs[b], [rows_lo, col])
            hi = plsc.load_gather(in_bufs[b], [rows_hi, col])
            out_bufs[b][pl.ds(i * D, LANES)] = lo
            out_bufs[b][pl.ds(i * D + LANES, LANES)] = hi
            return carry

        lax.fori_loop(0, n_items, body, 0, unroll=8)

    def transpose_full(b):
        transpose_n(b, CW)

    def transpose_tail(b):
        transpose_n(b, TAIL_N)

    fire_in(wid, 0)

    def loop_body(k, carry):
        b = k % 2

        def do(b):
            c = wid + k * NW

            @pl.when(c < N_FULL + 1)
            def _():
                @pl.when(c + NW < N_FULL + 1)
                def _():
                    fire_in(c + NW, 1 - b)

                drain_in(c, b)

                @pl.when(k >= 2)
                def _():
                    drain_out(c - 2 * NW, b)

                @pl.when(c < N_FULL)
                def _():
                    transpose_full(b)

                @pl.when(c == N_FULL)
                def _():
                    transpose_tail(b)

                fire_out(c, b)

        @pl.when(b == 0)
        def _():
            do(0)

        @pl.when(b == 1)
        def _():
            do(1)

        return carry

    lax.fori_loop(0, n_steps, loop_body, 0)

    for k in (n_steps - 2, n_steps - 1):
        c = wid + k * NW
        b = k % 2

        @pl.when(c < N_FULL + 1)
        def _():
            drain_out(c, b)


def _fix_padding(idx_v, rows_v):
    def detect_body(g, acc):
        v = idx_v[pl.ds(g * LANES, LANES)]
        return jnp.minimum(acc, v)

    min_idx = lax.fori_loop(
        0, CHUNK // LANES, detect_body,
        jnp.full((LANES,), 1, jnp.int32), unroll=8,
    )
    n_pad = jnp.min(min_idx)

    @pl.when(n_pad == 0)
    def _():
        def mask_body(g, carry):
            v = idx_v[pl.ds(g * LANES, LANES)]
            fmask = jnp.where(v == 0, 0.0, 1.0).astype(jnp.float32)
            for r in range(LANES):
                m = lax.gather(
                    fmask, jnp.full((LANES, 1), r, jnp.int32), _DNUMS, (1,),
                    mode=lax.GatherScatterMode.PROMISE_IN_BOUNDS,
                )
                rr = g * LANES + r
                rows_v[rr, pl.ds(0, LANES)] = rows_v[rr, pl.ds(0, LANES)] * m
                rows_v[rr, pl.ds(LANES, LANES)] = (
                    rows_v[rr, pl.ds(LANES, LANES)] * m
                )
            return carry

        lax.fori_loop(0, CHUNK // LANES, mask_body, 0)


def _emb_body(idx_hbm, tab_hbm, out_hbm,
              idx0_v, idx1_v, rows0_v, rows1_v,
              sem_g0, sem_g1, sem_o0, sem_o1):
    wid = lax.axis_index("s") * NC + lax.axis_index("c")
    n_rows = idx_hbm.shape[0]
    rows_per_w = n_rows // NW
    n_chunks = rows_per_w // CHUNK
    row0 = pl.multiple_of(wid * rows_per_w, 8)

    idx_bufs = (idx0_v, idx1_v)
    rows_bufs = (rows0_v, rows1_v)
    sems_g = (sem_g0, sem_g1)
    sems_o = (sem_o0, sem_o1)

    def stage_and_fire(c, b):
        rbase = pl.multiple_of(row0 + c * CHUNK, 8)
        pltpu.sync_copy(idx_hbm.at[pl.ds(rbase, CHUNK)], idx_bufs[b])
        pltpu.async_copy(tab_hbm.at[idx_bufs[b]], rows_bufs[b], sems_g[b])

    def drain_gathers(c, b):
        rbase = pl.multiple_of(row0 + c * CHUNK, 8)
        pltpu.make_async_copy(
            out_hbm.at[pl.ds(rbase, CHUNK), pl.ds(0, D)],
            rows_bufs[b],
            sems_g[b],
        ).wait()

    def fire_out(c, b):
        rbase = pl.multiple_of(row0 + c * CHUNK, 8)
        pltpu.async_copy(
            rows_bufs[b],
            out_hbm.at[pl.ds(rbase, CHUNK), pl.ds(0, D)],
            sems_o[b],
        )

    def drain_out(c, b):
        rbase = pl.multiple_of(row0 + c * CHUNK, 8)
        pltpu.make_async_copy(
            rows_bufs[b],
            out_hbm.at[pl.ds(rbase, CHUNK), pl.ds(0, D)],
            sems_o[b],
        ).wait()

    def chunk_step(c, b, p):
        ob = 1 - b

        @pl.when(c + 1 < n_chunks)
        def _():
            @pl.when(p > 0)
            def _():
                drain_out(c - 1, ob)

            stage_and_fire(c + 1, ob)

        drain_gathers(c, b)
        _fix_padding(idx_bufs[b], rows_bufs[b])
        fire_out(c, b)

    stage_and_fire(0, 0)

    def pair_body(p, carry):
        chunk_step(2 * p, 0, p)
        chunk_step(2 * p + 1, 1, p + 1)
        return carry

    lax.fori_loop(0, n_chunks // 2, pair_body, 0)
    drain_out(n_chunks - 2, 0)
    drain_out(n_chunks - 1, 1)


def kernel(x, weight):
    b, s = x.shape
    n_rows = b * s
    idx = x.reshape(n_rows).astype(jnp.int32)
    mesh = plsc.VectorSubcoreMesh(core_axis_name="c", subcore_axis_name="s")

    fmt = functools.partial(
        pl.kernel,
        mesh=mesh,
        out_type=jax.ShapeDtypeStruct((TBL_FLAT,), jnp.float32),
        scratch_types=[
            pltpu.VMEM((D, CW + 1), jnp.float32),
            pltpu.VMEM((D, CW + 1), jnp.float32),
            pltpu.VMEM((CW * D,), jnp.float32),
            pltpu.VMEM((CW * D,), jnp.float32),
            pltpu.SemaphoreType.DMA,
            pltpu.SemaphoreType.DMA,
            pltpu.SemaphoreType.DMA,
            pltpu.SemaphoreType.DMA,
        ],
        compiler_params=pltpu.CompilerParams(
            use_tc_tiling_on_sc=True, needs_layout_passes=False
        ),
    )(_format_body)
    tab = fmt(weight.T).reshape(N_ROWS_PAD, D)

    gat = functools.partial(
        pl.kernel,
        mesh=mesh,
        out_type=jax.ShapeDtypeStruct((n_rows, 128), jnp.float32),
        scratch_types=[
            pltpu.VMEM((CHUNK,), jnp.int32),
            pltpu.VMEM((CHUNK,), jnp.int32),
            pltpu.VMEM((CHUNK, D), jnp.float32),
            pltpu.VMEM((CHUNK, D), jnp.float32),
            pltpu.SemaphoreType.DMA,
            pltpu.SemaphoreType.DMA,
            pltpu.SemaphoreType.DMA,
            pltpu.SemaphoreType.DMA,
        ],
        compiler_params=pltpu.CompilerParams(
            use_tc_tiling_on_sc=False, needs_layout_passes=False
        ),
    )(_emb_body)
    out = gat(idx, tab)
    return out[:, :D].reshape(b, s, D)


# speedup vs baseline: 1.2499x; 1.2499x over previous
import functools

import jax
import jax.numpy as jnp
from jax import lax
from jax.experimental import pallas as pl
from jax.experimental.pallas import tpu as pltpu
from jax.experimental.pallas import tpu_sc as plsc

LANES = 16
NC = 2
NS = 16
NW = NC * NS

CHUNK = 1280
D = 32

_DNUMS = lax.GatherDimensionNumbers(
    offset_dims=(), collapsed_slice_dims=(0,), start_index_map=(0,)
)


def _fix_padding(idx_v, rows_v):
    def detect_body(g, acc):
        v = idx_v[pl.ds(g * LANES, LANES)]
        return jnp.minimum(acc, v)

    min_idx = lax.fori_loop(
        0,
        CHUNK // LANES,
        detect_body,
        jnp.full((LANES,), 1, jnp.int32),
        unroll=8,
    )
    n_pad = jnp.min(min_idx)

    @pl.when(n_pad == 0)
    def _():
        def mask_body(g, carry):
            v = idx_v[pl.ds(g * LANES, LANES)]
            fmask = jnp.where(v == 0, 0.0, 1.0).astype(jnp.float32)
            for r in range(LANES):
                m = lax.gather(
                    fmask,
                    jnp.full((LANES, 1), r, jnp.int32),
                    _DNUMS,
                    (1,),
                    mode=lax.GatherScatterMode.PROMISE_IN_BOUNDS,
                )
                rr = g * LANES + r
                rows_v[rr, pl.ds(0, LANES)] = rows_v[rr, pl.ds(0, LANES)] * m
                rows_v[rr, pl.ds(LANES, LANES)] = (
                    rows_v[rr, pl.ds(LANES, LANES)] * m
                )
            return carry

        lax.fori_loop(0, CHUNK // LANES, mask_body, 0)


def _emb_body(idx_hbm, tab_hbm, out_hbm,
              idx0_v, idx1_v, rows0_v, rows1_v,
              sem_g0, sem_g1, sem_o0, sem_o1):
    wid = lax.axis_index("s") * NC + lax.axis_index("c")
    n_rows = idx_hbm.shape[0]
    rows_per_w = n_rows // NW
    n_chunks = rows_per_w // CHUNK
    row0 = pl.multiple_of(wid * rows_per_w, 8)

    idx_bufs = (idx0_v, idx1_v)
    rows_bufs = (rows0_v, rows1_v)
    sems_g = (sem_g0, sem_g1)
    sems_o = (sem_o0, sem_o1)

    def stage_and_fire(c, b):
        rbase = pl.multiple_of(row0 + c * CHUNK, 8)
        pltpu.sync_copy(idx_hbm.at[pl.ds(rbase, CHUNK)], idx_bufs[b])
        pltpu.async_copy(tab_hbm.at[idx_bufs[b]], rows_bufs[b], sems_g[b])

    def drain_gathers(c, b):
        rbase = pl.multiple_of(row0 + c * CHUNK, 8)
        pltpu.make_async_copy(
            out_hbm.at[pl.ds(rbase, CHUNK), pl.ds(0, D)],
            rows_bufs[b],
            sems_g[b],
        ).wait()

    def fire_out(c, b):
        rbase = pl.multiple_of(row0 + c * CHUNK, 8)
        pltpu.async_copy(
            rows_bufs[b],
            out_hbm.at[pl.ds(rbase, CHUNK), pl.ds(0, D)],
            sems_o[b],
        )

    def drain_out(c, b):
        rbase = pl.multiple_of(row0 + c * CHUNK, 8)
        pltpu.make_async_copy(
            rows_bufs[b],
            out_hbm.at[pl.ds(rbase, CHUNK), pl.ds(0, D)],
            sems_o[b],
        ).wait()

    def chunk_step(c, b, p):
        ob = 1 - b

        @pl.when(c + 1 < n_chunks)
        def _():
            @pl.when(p > 0)
            def _():
                drain_out(c - 1, ob)

            stage_and_fire(c + 1, ob)

        drain_gathers(c, b)
        _fix_padding(idx_bufs[b], rows_bufs[b])
        fire_out(c, b)

    stage_and_fire(0, 0)

    def pair_body(p, carry):
        chunk_step(2 * p, 0, p)
        chunk_step(2 * p + 1, 1, p + 1)
        return carry

    lax.fori_loop(0, n_chunks // 2, pair_body, 0)
    drain_out(n_chunks - 2, 0)
    drain_out(n_chunks - 1, 1)


def kernel(x, weight):
    b, s = x.shape
    n_rows = b * s
    idx = x.reshape(n_rows).astype(jnp.int32)

    mesh = plsc.VectorSubcoreMesh(core_axis_name="c", subcore_axis_name="s")
    fn = functools.partial(
        pl.kernel,
        mesh=mesh,
        out_type=jax.ShapeDtypeStruct((n_rows, 128), jnp.float32),
        scratch_types=[
            pltpu.VMEM((CHUNK,), jnp.int32),
            pltpu.VMEM((CHUNK,), jnp.int32),
            pltpu.VMEM((CHUNK, D), jnp.float32),
            pltpu.VMEM((CHUNK, D), jnp.float32),
            pltpu.SemaphoreType.DMA,
            pltpu.SemaphoreType.DMA,
            pltpu.SemaphoreType.DMA,
            pltpu.SemaphoreType.DMA,
        ],
        compiler_params=pltpu.CompilerParams(
            use_tc_tiling_on_sc=False, needs_layout_passes=False
        ),
    )(_emb_body)
    out = fn(idx, weight)
    return out[:, :D].reshape(b, s, D)
